# Initial kernel scaffold; baseline (speedup 1.0000x reference)
import jax
import jax.numpy as jnp
from jax.experimental import pallas as pl


def kernel(x_nchw, w1k, b1k, w2k, b2k, whk, bhk, wok, bok):
    raise NotImplementedError("write your pallas kernel here")


# speedup vs baseline: 40.1123x; 40.1123x over previous
import numpy as np

import jax
import jax.numpy as jnp
from jax.experimental import pallas as pl
from jax.experimental.pallas import tpu as pltpu

NB = 256
HP, WP = 33, 32
C = 40
H2P = 18


def _tap_index(n_r, n_w, n_dh, n_wi):
    r = np.arange(n_r)[:, None, None, None]
    w = np.arange(n_w)[None, :, None, None]
    dh = np.arange(n_dh)[None, None, :, None]
    wi = np.arange(n_wi)[None, None, None, :]
    kh = dh - r
    kw = wi - w
    valid = (kh >= 0) & (kh < 5) & (kw >= 0) & (kw < 5)
    return np.where(valid, kh * 5 + kw, 25).astype(np.int32)


_TAP1 = _tap_index(4, 28, 8, WP)
_TAP2 = _tap_index(2, 14, 6, H2P)


def _body(x_ref, w1t_ref, b1_ref, w2t_ref, b2_ref, whm_ref, bh_ref,
          wot_ref, bo_ref, o_ref, p1_ref, p2_ref):

    p1_ref[...] = jnp.zeros_like(p1_ref)

    w1t = w1t_ref[...]
    b1 = b1_ref[...]
    for s in range(7):
        xs = x_ref[pl.ds(128 * s, 256), :]
        v = jnp.dot(w1t, xs, preferred_element_type=jnp.float32)
        v = v.astype(jnp.bfloat16).reshape(2, 2, 28, C, NB)
        vr = jnp.maximum(v[:, 0], v[:, 1])
        vr = vr.reshape(2, 14, 2, C, NB)
        vw = jnp.maximum(vr[:, :, 0], vr[:, :, 1])
        p1 = jnp.maximum(vw + b1.astype(jnp.bfloat16), 0.0)
        p1_ref[2 + 2 * s:4 + 2 * s, 2:16, :, :] = p1

    w2t = w2t_ref[...]
    b2 = b2_ref[...]
    for t in range(7):
        x2 = p1_ref[2 * t:2 * t + 6, :, :, :].reshape(6 * H2P * C, NB)
        v = jnp.dot(w2t, x2, preferred_element_type=jnp.float32)
        v = v.astype(jnp.bfloat16).reshape(2, 14, C, NB)
        vr = jnp.maximum(v[0], v[1]).reshape(7, 2, C, NB)
        vw = jnp.maximum(vr[:, 0], vr[:, 1])
        p2 = jnp.maximum(vw + b2.astype(jnp.bfloat16), 0.0)
        p2_ref[pl.ds(7 * t, 7), :, :] = p2

    flat = p2_ref[...].reshape(49 * C, NB)
    h = jnp.dot(whm_ref[...], flat, preferred_element_type=jnp.float32)
    h = jnp.maximum(h + bh_ref[...], 0.0)
    o = jnp.dot(wot_ref[...], h.astype(jnp.bfloat16),
                preferred_element_type=jnp.float32)
    o_ref[...] = o + bo_ref[...]


def kernel(x_nchw, w1k, b1k, w2k, b2k, whk, bhk, wok, bok):
    B = x_nchw.shape[0]
    Bp = ((B + NB - 1) // NB) * NB
    HID = whk.shape[-1]

    x = x_nchw.reshape(B, 28, 28)
    x = jnp.pad(x, ((0, Bp - B), (2, 3), (2, 2)))
    xt = x.reshape(Bp, HP * WP).T.astype(jnp.bfloat16)

    w1pad = jnp.concatenate([w1k, jnp.zeros((1, C), w1k.dtype)], axis=0)
    w1t = w1pad[_TAP1]
    w1t = w1t.transpose(0, 1, 4, 2, 3).reshape(4 * 28 * C, 8 * WP)
    w1t = w1t.astype(jnp.bfloat16)

    w2pad = jnp.concatenate([w2k, jnp.zeros((1, C, C), w2k.dtype)], axis=0)
    w2t = w2pad[_TAP2]
    w2t = w2t.transpose(0, 1, 5, 2, 3, 4).reshape(2 * 14 * C, 6 * H2P * C)
    w2t = w2t.astype(jnp.bfloat16)

    whm = whk.transpose(2, 0, 1).reshape(HID, 49 * C).astype(jnp.bfloat16)
    wot = wok.T.astype(jnp.bfloat16)

    out = pl.pallas_call(
        _body,
        out_shape=jax.ShapeDtypeStruct((10, Bp), jnp.float32),
        grid=(Bp // NB,),
        in_specs=[
            pl.BlockSpec((HP * WP, NB), lambda g: (0, g)),
            pl.BlockSpec(w1t.shape, lambda g: (0, 0)),
            pl.BlockSpec((C, 1), lambda g: (0, 0)),
            pl.BlockSpec(w2t.shape, lambda g: (0, 0)),
            pl.BlockSpec((C, 1), lambda g: (0, 0)),
            pl.BlockSpec((HID, 49 * C), lambda g: (0, 0)),
            pl.BlockSpec((HID, 1), lambda g: (0, 0)),
            pl.BlockSpec((10, HID), lambda g: (0, 0)),
            pl.BlockSpec((10, 1), lambda g: (0, 0)),
        ],
        out_specs=pl.BlockSpec((10, NB), lambda g: (0, g)),
        scratch_shapes=[
            pltpu.VMEM((H2P, H2P, C, NB), jnp.bfloat16),
            pltpu.VMEM((49, C, NB), jnp.bfloat16),
        ],
        compiler_params=pltpu.CompilerParams(
            dimension_semantics=("parallel",),
            vmem_limit_bytes=100 * 1024 * 1024,
        ),
    )(xt, w1t, b1k.reshape(C, 1), w2t, b2k.reshape(C, 1),
      whm, bhk.reshape(HID, 1), wot, bok.reshape(10, 1))
    return out.T[:B]


# speedup vs baseline: 43.0361x; 1.0729x over previous
import numpy as np

import jax
import jax.numpy as jnp
from jax.experimental import pallas as pl
from jax.experimental.pallas import tpu as pltpu

NB = 512
HP, WP = 33, 32
C = 40
H2P = 18


def _tap_index(n_r, n_w, n_dh, n_wi):
    r = np.arange(n_r)[:, None, None, None]
    w = np.arange(n_w)[None, :, None, None]
    dh = np.arange(n_dh)[None, None, :, None]
    wi = np.arange(n_wi)[None, None, None, :]
    kh = dh - r
    kw = wi - w
    valid = (kh >= 0) & (kh < 5) & (kw >= 0) & (kw < 5)
    return np.where(valid, kh * 5 + kw, 25).astype(np.int32)


_TAP1 = _tap_index(4, 28, 8, WP)
_TAP2 = _tap_index(1, 14, 5, H2P)


def _body(x_ref, w1t_ref, b1_ref, w2t_ref, b2_ref, whm_ref, bh_ref,
          wot_ref, bo_ref, o_ref, xp_ref, p1_ref, p2_ref):

    xp_ref[...] = jnp.zeros_like(xp_ref)
    p1_ref[...] = jnp.zeros_like(p1_ref)
    for i in range(28):
        xp_ref[2 + i, 2:30, :] = x_ref[pl.ds(28 * i, 28), :]

    w1t = w1t_ref[...]
    b1 = b1_ref[...].astype(jnp.bfloat16)
    for s in range(7):
        xs = xp_ref[4 * s:4 * s + 8, :, :].reshape(256, NB)
        v = jnp.dot(w1t, xs, preferred_element_type=jnp.float32)
        v = v.astype(jnp.bfloat16).reshape(2, 2, 28, C, NB)
        vr = jnp.maximum(v[:, 0], v[:, 1])
        vr = vr.reshape(2, 14, 2, C, NB)
        vw = jnp.maximum(vr[:, :, 0], vr[:, :, 1])
        p1 = jnp.maximum(vw + b1, 0.0)
        p1_ref[2 + 2 * s:4 + 2 * s, 2:16, :, :] = p1

    w2t = w2t_ref[...]
    b2 = b2_ref[...].astype(jnp.bfloat16)
    for u in range(7):
        xa = p1_ref[2 * u:2 * u + 5, :, :, :].reshape(5 * H2P * C, NB)
        xb = p1_ref[2 * u + 1:2 * u + 6, :, :, :].reshape(5 * H2P * C, NB)
        va = jnp.dot(w2t, xa, preferred_element_type=jnp.float32)
        vb = jnp.dot(w2t, xb, preferred_element_type=jnp.float32)
        vr = jnp.maximum(va, vb).astype(jnp.bfloat16).reshape(7, 2, C, NB)
        vw = jnp.maximum(vr[:, 0], vr[:, 1])
        p2 = jnp.maximum(vw + b2, 0.0)
        p2_ref[pl.ds(7 * u, 7), :, :] = p2

    flat = p2_ref[...].reshape(49 * C, NB)
    h = jnp.dot(whm_ref[...], flat, preferred_element_type=jnp.float32)
    h = jnp.maximum(h + bh_ref[...], 0.0)
    o = jnp.dot(wot_ref[...], h.astype(jnp.bfloat16),
                preferred_element_type=jnp.float32)
    o_ref[...] = o + bo_ref[...]


def kernel(x_nchw, w1k, b1k, w2k, b2k, whk, bhk, wok, bok):
    B = x_nchw.shape[0]
    Bp = ((B + NB - 1) // NB) * NB
    HID = whk.shape[-1]

    x = x_nchw.reshape(B, 28 * 28)
    x = jnp.pad(x, ((0, Bp - B), (0, 0)))
    xt = x.T.astype(jnp.bfloat16)

    w1pad = jnp.concatenate([w1k, jnp.zeros((1, C), w1k.dtype)], axis=0)
    w1t = w1pad[_TAP1]
    w1t = w1t.transpose(0, 1, 4, 2, 3).reshape(4 * 28 * C, 8 * WP)
    w1t = w1t.astype(jnp.bfloat16)

    w2pad = jnp.concatenate([w2k, jnp.zeros((1, C, C), w2k.dtype)], axis=0)
    w2t = w2pad[_TAP2]
    w2t = w2t.transpose(0, 1, 5, 2, 3, 4).reshape(14 * C, 5 * H2P * C)
    w2t = w2t.astype(jnp.bfloat16)

    whm = whk.transpose(2, 0, 1).reshape(HID, 49 * C).astype(jnp.bfloat16)
    wot = wok.T.astype(jnp.bfloat16)

    out = pl.pallas_call(
        _body,
        out_shape=jax.ShapeDtypeStruct((10, Bp), jnp.float32),
        grid=(Bp // NB,),
        in_specs=[
            pl.BlockSpec((28 * 28, NB), lambda g: (0, g)),
            pl.BlockSpec(w1t.shape, lambda g: (0, 0)),
            pl.BlockSpec((C, 1), lambda g: (0, 0)),
            pl.BlockSpec(w2t.shape, lambda g: (0, 0)),
            pl.BlockSpec((C, 1), lambda g: (0, 0)),
            pl.BlockSpec((HID, 49 * C), lambda g: (0, 0)),
            pl.BlockSpec((HID, 1), lambda g: (0, 0)),
            pl.BlockSpec((10, HID), lambda g: (0, 0)),
            pl.BlockSpec((10, 1), lambda g: (0, 0)),
        ],
        out_specs=pl.BlockSpec((10, NB), lambda g: (0, g)),
        scratch_shapes=[
            pltpu.VMEM((HP, WP, NB), jnp.bfloat16),
            pltpu.VMEM((H2P, H2P, C, NB), jnp.bfloat16),
            pltpu.VMEM((49, C, NB), jnp.bfloat16),
        ],
        compiler_params=pltpu.CompilerParams(
            dimension_semantics=("parallel",),
            vmem_limit_bytes=100 * 1024 * 1024,
        ),
    )(xt, w1t, b1k.reshape(C, 1), w2t, b2k.reshape(C, 1),
      whm, bhk.reshape(HID, 1), wot, bok.reshape(10, 1))
    return out.T[:B]
